# baseline (speedup 1.0000x reference)
import functools

import jax
import jax.numpy as jnp
from jax import lax
from jax.experimental import pallas as pl
from jax.experimental.pallas import tpu as pltpu
from jax.experimental.pallas import tpu_sc as plsc

NC = 2
NS = 16
NW = NC * NS
CHUNK = 128
NBUF = 6
D = 3


@functools.lru_cache(maxsize=None)
def _build(n_rows: int, hidden: int, total: int):
    assert total % (NW * CHUNK) == 0
    per_w = total // NW
    n_chunk = per_w // CHUNK
    assert n_chunk > NBUF

    mesh = plsc.VectorSubcoreMesh(core_axis_name="c", subcore_axis_name="s")

    @functools.partial(
        pl.kernel,
        mesh=mesh,
        out_type=[
            jax.ShapeDtypeStruct((total, hidden), jnp.float32),
            jax.ShapeDtypeStruct((NW, per_w), jnp.float32),
        ],
        scratch_types=[
            pltpu.VMEM((n_chunk, CHUNK), jnp.int32),
            pltpu.VMEM((NBUF, CHUNK, hidden), jnp.float32),
            pltpu.VMEM((per_w,), jnp.float32),
            pltpu.SemaphoreType.DMA,
            pltpu.SemaphoreType.DMA,
            pltpu.SemaphoreType.DMA,
        ],
    )
    def emb_gather(w_hbm, b_hbm, idx_hbm, outw_hbm, outb_hbm,
                   idx_v, rows_v, b_v, gsem, bsem, wsem):
        wid = lax.axis_index("s") * NC + lax.axis_index("c")
        base = wid * per_w
        pltpu.sync_copy(idx_hbm.at[wid], idx_v)

        for t in range(D):
            pltpu.async_copy(w_hbm.at[idx_v.at[t]], rows_v.at[t], gsem)
            pltpu.async_copy(
                b_hbm.at[idx_v.at[t]], b_v.at[pl.ds(t * CHUNK, CHUNK)], bsem)

        def body(j, carry):
            t = lax.rem(j, NBUF)
            pltpu.make_async_copy(
                w_hbm.at[pl.ds(0, CHUNK)], rows_v.at[t], gsem).wait()
            off = base + j * CHUNK
            pltpu.async_copy(rows_v.at[t], outw_hbm.at[pl.ds(off, CHUNK)], wsem)

            @pl.when(j >= NBUF - D)
            def _():
                pltpu.make_async_copy(
                    rows_v.at[0], outw_hbm.at[pl.ds(base, CHUNK)], wsem).wait()

            jn = j + D

            @pl.when(jn < n_chunk)
            def _():
                tn = lax.rem(jn, NBUF)
                pltpu.async_copy(w_hbm.at[idx_v.at[jn]], rows_v.at[tn], gsem)
                pltpu.async_copy(
                    b_hbm.at[idx_v.at[jn]], b_v.at[pl.ds(jn * CHUNK, CHUNK)],
                    bsem)

            return carry

        lax.fori_loop(0, n_chunk, body, 0)

        for _ in range(NBUF - D):
            pltpu.make_async_copy(
                rows_v.at[0], outw_hbm.at[pl.ds(base, CHUNK)], wsem).wait()
        pltpu.make_async_copy(b_hbm.at[pl.ds(0, per_w)], b_v, bsem).wait()
        pltpu.sync_copy(b_v, outb_hbm.at[wid])

    return emb_gather


def kernel(W, b, output_indices):
    n_rows, hidden = W.shape
    bsz, sl = output_indices.shape
    total = bsz * sl
    idx3 = output_indices.reshape(NW, total // (NW * CHUNK), CHUNK)
    b_flat = b.reshape(-1)
    outw, outb = _build(n_rows, hidden, total)(W, b_flat, idx3)
    return (outw.reshape(bsz, sl, hidden), outb.reshape(bsz, sl, 1))


# speedup vs baseline: 1.6394x; 1.6394x over previous
import functools

import jax
import jax.numpy as jnp
from jax import lax
from jax.experimental import pallas as pl
from jax.experimental.pallas import tpu as pltpu
from jax.experimental.pallas import tpu_sc as plsc

NC = 2
NS = 16
NW = NC * NS
BCHUNK = 128
NBUF = 8
D = 4


@functools.lru_cache(maxsize=None)
def _build(n_rows: int, hidden: int, bsz: int, sl: int):
    assert bsz % NW == 0
    bat_w = bsz // NW
    per_w = bat_w * sl
    assert per_w % BCHUNK == 0
    nb_chunk = per_w // BCHUNK
    assert bat_w > NBUF
    assert per_w == sl * hidden

    mesh = plsc.VectorSubcoreMesh(core_axis_name="c", subcore_axis_name="s")

    @functools.partial(
        pl.kernel,
        mesh=mesh,
        out_type=[
            jax.ShapeDtypeStruct((bsz, sl, hidden), jnp.float32),
            jax.ShapeDtypeStruct((NW, per_w), jnp.float32),
        ],
        scratch_types=[
            pltpu.VMEM((bat_w, sl), jnp.int32),
            pltpu.VMEM((nb_chunk, BCHUNK), jnp.int32),
            pltpu.VMEM((NBUF, sl, hidden), jnp.float32),
            pltpu.VMEM((per_w,), jnp.float32),
            pltpu.SemaphoreType.DMA,
            pltpu.SemaphoreType.DMA,
            pltpu.SemaphoreType.DMA,
        ],
    )
    def emb_gather(w_hbm, b_hbm, idxw_hbm, idxb_hbm, outw_hbm, outb_hbm,
                   idxw_v, idxb_v, rows_v, b_v, gsem, bsem, wsem):
        wid = lax.axis_index("s") * NC + lax.axis_index("c")
        bat0 = wid * bat_w
        pltpu.sync_copy(idxw_hbm.at[wid], idxw_v)
        pltpu.sync_copy(idxb_hbm.at[wid], idxb_v)

        for t in range(nb_chunk):
            pltpu.async_copy(
                b_hbm.at[idxb_v.at[t]], b_v.at[pl.ds(t * BCHUNK, BCHUNK)],
                bsem)

        for t in range(D):
            pltpu.async_copy(w_hbm.at[idxw_v.at[t]], rows_v.at[t], gsem)

        def body(j, carry):
            t = lax.rem(j, NBUF)
            pltpu.make_async_copy(
                b_hbm.at[pl.ds(0, per_w)], b_v, gsem).wait()
            pltpu.async_copy(rows_v.at[t], outw_hbm.at[bat0 + j], wsem)

            @pl.when(j >= NBUF - D)
            def _():
                pltpu.make_async_copy(
                    rows_v.at[0], outw_hbm.at[0], wsem).wait()

            jn = j + D

            @pl.when(jn < bat_w)
            def _():
                tn = lax.rem(jn, NBUF)
                pltpu.async_copy(w_hbm.at[idxw_v.at[jn]], rows_v.at[tn], gsem)

            return carry

        lax.fori_loop(0, bat_w, body, 0)

        for _ in range(NBUF - D):
            pltpu.make_async_copy(
                rows_v.at[0], outw_hbm.at[0], wsem).wait()
        pltpu.make_async_copy(b_hbm.at[pl.ds(0, per_w)], b_v, bsem).wait()
        pltpu.sync_copy(b_v, outb_hbm.at[wid])

    return emb_gather


def kernel(W, b, output_indices):
    n_rows, hidden = W.shape
    bsz, sl = output_indices.shape
    total = bsz * sl
    idxw = output_indices.reshape(NW, bsz // NW, sl)
    idxb = output_indices.reshape(NW, total // (NW * BCHUNK), BCHUNK)
    b_flat = b.reshape(-1)
    outw, outb = _build(n_rows, hidden, bsz, sl)(W, b_flat, idxw, idxb)
    return (outw, outb.reshape(bsz, sl, 1))


# speedup vs baseline: 2.8824x; 1.7582x over previous
import functools

import jax
import jax.numpy as jnp
from jax import lax
from jax.experimental import pallas as pl
from jax.experimental.pallas import tpu as pltpu
from jax.experimental.pallas import tpu_sc as plsc

NC = 2
NS = 16
NW = NC * NS
NBUF = 6
D = 3


@functools.lru_cache(maxsize=None)
def _build(n_rows: int, hidden: int, bsz: int, sl: int):
    assert bsz % NW == 0
    bat_w = bsz // NW
    assert bat_w % 8 == 0 and bat_w <= 128
    assert sl > NBUF

    mesh = plsc.VectorSubcoreMesh(core_axis_name="c", subcore_axis_name="s")

    @functools.partial(
        pl.kernel,
        mesh=mesh,
        out_type=[
            jax.ShapeDtypeStruct((sl, bsz, hidden), jnp.float32),
            jax.ShapeDtypeStruct((NW, sl, bat_w), jnp.float32),
        ],
        scratch_types=[
            pltpu.VMEM((sl, bat_w), jnp.int32),
            pltpu.VMEM((NBUF, bat_w, hidden), jnp.float32),
            pltpu.VMEM((sl, bat_w), jnp.float32),
            pltpu.SemaphoreType.DMA,
            pltpu.SemaphoreType.DMA,
            pltpu.SemaphoreType.DMA,
        ],
    )
    def emb_gather(w_hbm, b_hbm, idx_hbm, outw_hbm, outb_hbm,
                   idx_v, rows_v, b_v, gsem, bsem, wsem):
        wid = lax.axis_index("s") * NC + lax.axis_index("c")
        bat0 = wid * bat_w
        pltpu.sync_copy(idx_hbm.at[wid], idx_v)

        for j in range(sl):
            pltpu.async_copy(b_hbm.at[idx_v.at[j]], b_v.at[j], bsem)

        for t in range(D):
            pltpu.async_copy(w_hbm.at[idx_v.at[t]], rows_v.at[t], gsem)

        def body(j, carry):
            t = lax.rem(j, NBUF)
            pltpu.make_async_copy(
                w_hbm.at[pl.ds(0, bat_w)], rows_v.at[t], gsem).wait()
            pltpu.async_copy(
                rows_v.at[t], outw_hbm.at[j, pl.ds(bat0, bat_w)], wsem)

            @pl.when(j >= NBUF - D)
            def _():
                pltpu.make_async_copy(
                    rows_v.at[0], outw_hbm.at[0, pl.ds(0, bat_w)],
                    wsem).wait()

            jn = j + D

            @pl.when(jn < sl)
            def _():
                tn = lax.rem(jn, NBUF)
                pltpu.async_copy(w_hbm.at[idx_v.at[jn]], rows_v.at[tn], gsem)

            return carry

        lax.fori_loop(0, sl, body, 0)

        for _ in range(NBUF - D):
            pltpu.make_async_copy(
                rows_v.at[0], outw_hbm.at[0, pl.ds(0, bat_w)], wsem).wait()
        for _ in range(sl):
            pltpu.make_async_copy(
                b_hbm.at[pl.ds(0, bat_w)], b_v.at[0], bsem).wait()
        pltpu.sync_copy(b_v, outb_hbm.at[wid])

    return emb_gather


def kernel(W, b, output_indices):
    n_rows, hidden = W.shape
    bsz, sl = output_indices.shape
    idxT = output_indices.reshape(NW, bsz // NW, sl).transpose(0, 2, 1)
    b_flat = b.reshape(-1)
    outT, outb = _build(n_rows, hidden, bsz, sl)(W, b_flat, idxT)
    w_act = outT.transpose(1, 0, 2)
    b_act = outb.transpose(0, 2, 1).reshape(bsz, sl, 1)
    return (w_act, b_act)


# speedup vs baseline: 2.9468x; 1.0223x over previous
import functools

import jax
import jax.numpy as jnp
from jax import lax
from jax.experimental import pallas as pl
from jax.experimental.pallas import tpu as pltpu
from jax.experimental.pallas import tpu_sc as plsc

NC = 2
NS = 16
NW = NC * NS
NBUF = 6
D = 3


@functools.lru_cache(maxsize=None)
def _build(n_rows: int, hidden: int, bsz: int, sl: int):
    assert bsz % NW == 0
    bat_w = bsz // NW
    assert bat_w % 8 == 0 and bat_w <= 128
    assert sl > NBUF

    mesh = plsc.VectorSubcoreMesh(core_axis_name="c", subcore_axis_name="s")

    @functools.partial(
        pl.kernel,
        mesh=mesh,
        out_type=[
            jax.ShapeDtypeStruct((sl, bsz, hidden), jnp.float32),
            jax.ShapeDtypeStruct((NW, sl, bat_w), jnp.float32),
        ],
        scratch_types=[
            pltpu.VMEM((sl, bat_w), jnp.int32),
            pltpu.VMEM((NBUF, bat_w, hidden), jnp.float32),
            pltpu.VMEM((sl, bat_w), jnp.float32),
            pltpu.SemaphoreType.DMA,
            pltpu.SemaphoreType.DMA,
            pltpu.SemaphoreType.DMA,
        ],
    )
    def emb_gather(w_hbm, b_hbm, idx_hbm, outw_hbm, outb_hbm,
                   idx_v, rows_v, b_v, gsem, bsem, wsem):
        wid = lax.axis_index("s") * NC + lax.axis_index("c")
        bat0 = wid * bat_w
        pltpu.sync_copy(idx_hbm.at[wid], idx_v)

        for t in range(D):
            pltpu.async_copy(w_hbm.at[idx_v.at[t]], rows_v.at[t], gsem)

        def body(j, carry):
            t = lax.rem(j, NBUF)
            pltpu.async_copy(b_hbm.at[idx_v.at[j]], b_v.at[j], bsem)
            pltpu.make_async_copy(
                w_hbm.at[pl.ds(0, bat_w)], rows_v.at[t], gsem).wait()
            pltpu.async_copy(
                rows_v.at[t], outw_hbm.at[j, pl.ds(bat0, bat_w)], wsem)

            @pl.when(j >= NBUF - D)
            def _():
                pltpu.make_async_copy(
                    rows_v.at[0], outw_hbm.at[0, pl.ds(0, bat_w)],
                    wsem).wait()

            jn = j + D

            @pl.when(jn < sl)
            def _():
                tn = lax.rem(jn, NBUF)
                pltpu.async_copy(w_hbm.at[idx_v.at[jn]], rows_v.at[tn], gsem)

            return carry

        lax.fori_loop(0, sl, body, 0)

        for _ in range(NBUF - D):
            pltpu.make_async_copy(
                rows_v.at[0], outw_hbm.at[0, pl.ds(0, bat_w)], wsem).wait()
        for _ in range(sl):
            pltpu.make_async_copy(
                b_hbm.at[pl.ds(0, bat_w)], b_v.at[0], bsem).wait()
        pltpu.sync_copy(b_v, outb_hbm.at[wid])

    return emb_gather


def kernel(W, b, output_indices):
    n_rows, hidden = W.shape
    bsz, sl = output_indices.shape
    idxT = output_indices.reshape(NW, bsz // NW, sl).transpose(0, 2, 1)
    b_flat = b.reshape(-1)
    outT, outb = _build(n_rows, hidden, bsz, sl)(W, b_flat, idxT)
    w_act = outT.transpose(1, 0, 2)
    b_act = outb.transpose(0, 2, 1).reshape(bsz, sl, 1)
    return (w_act, b_act)
